# baseline (speedup 1.0000x reference)
import functools

import jax
import jax.numpy as jnp
from jax import lax
from jax.experimental import pallas as pl
from jax.experimental.pallas import tpu as pltpu
from jax.experimental.pallas import tpu_sc as plsc

N = 10000
D = 128
E = 320000
MIN_NORM = 1e-15
MAXNORM = 1.0 - 4e-3

NC = 2
NS = 16
NW = NC * NS
L = 16

N_PAD = 10240
EPT = 10272
E_PAD = EPT * NW
CHUNK = 96
NCH = EPT // CHUNK
EPT1 = E_PAD // NS
NCH1 = EPT1 // CHUNK
ROWS_PT = N_PAD // NS


def _mesh():
    return plsc.VectorSubcoreMesh(
        core_axis_name="c", subcore_axis_name="s", num_cores=NC, num_subcores=NS
    )


def _hist_body(row_hbm, col_hbm, rc_out, dc_out, rbuf, cbuf, rc_priv, dc_priv):
    cid = lax.axis_index("c")
    sid = lax.axis_index("s")
    wid = sid * NC + cid

    zeros = jnp.zeros((L,), jnp.float32)

    def zero_body(i, _):
        rc_priv[pl.ds(i * L, L)] = zeros
        dc_priv[pl.ds(i * L, L)] = zeros
        return _

    lax.fori_loop(0, N_PAD // L, zero_body, 0)

    base = wid * EPT
    pltpu.sync_copy(row_hbm.at[pl.ds(base, EPT)], rbuf)
    pltpu.sync_copy(col_hbm.at[pl.ds(base, EPT)], cbuf)

    ones = jnp.full((L,), 1.0, jnp.float32)

    def body(i, _):
        r = rbuf[pl.ds(i * L, L)]
        c = cbuf[pl.ds(i * L, L)]
        plsc.addupdate_scatter(rc_priv, [r], ones)
        plsc.addupdate_scatter(dc_priv, [r], ones, mask=r == c)
        return _

    lax.fori_loop(0, EPT // L, body, 0)

    pltpu.sync_copy(rc_priv, rc_out.at[wid])
    pltpu.sync_copy(dc_priv, dc_out.at[wid])


def _hist(row_p, col_p):
    return pl.kernel(
        _hist_body,
        out_type=(
            jax.ShapeDtypeStruct((NW, N_PAD), jnp.float32),
            jax.ShapeDtypeStruct((NW, N_PAD), jnp.float32),
        ),
        mesh=_mesh(),
        compiler_params=pltpu.CompilerParams(needs_layout_passes=False),
        scratch_types=[
            pltpu.VMEM((EPT,), jnp.int32),
            pltpu.VMEM((EPT,), jnp.int32),
            pltpu.VMEM((N_PAD,), jnp.float32),
            pltpu.VMEM((N_PAD,), jnp.float32),
        ],
    )(row_p, col_p)


def _proj(v):
    n = jnp.maximum(jnp.sqrt(jnp.sum(v * v, axis=1, keepdims=True)), MIN_NORM)
    return jnp.where(n > MAXNORM, v * (MAXNORM / n), v)


def _dense_body(x_ref, w_ref, b_ref, rc_ref, dc_ref, u_ref, g_ref, sw_ref):
    i = pl.program_id(0)
    xb = x_ref[...]
    w = w_ref[...]
    h = lax.dot_general(xb, w, (((1,), (1,)), ((), ())),
                        preferred_element_type=jnp.float32)
    h = _proj(h)
    bias = b_ref[...]
    bn = jnp.maximum(jnp.sqrt(jnp.sum(bias * bias, axis=1, keepdims=True)),
                     MIN_NORM)
    b = jnp.tanh(bn) * bias / bn
    b = _proj(b)
    x2 = jnp.sum(h * h, axis=1, keepdims=True)
    y2 = jnp.sum(b * b, axis=1, keepdims=True)
    xy = jnp.sum(h * b, axis=1, keepdims=True)
    numer = (1.0 + 2.0 * xy + y2) * h + (1.0 - x2) * b
    denom = 1.0 + 2.0 * xy + x2 * y2
    h = numer / jnp.maximum(denom, MIN_NORM)
    h = _proj(h)
    s = 2.0 * h / (1.0 + jnp.sum(h * h, axis=1, keepdims=True))
    lamb = lax.rsqrt(jnp.maximum(1.0 - jnp.sum(s * s, axis=1, keepdims=True),
                                 MIN_NORM))
    rcs = jnp.sum(rc_ref[...], axis=1, keepdims=True)
    dcs = jnp.sum(dc_ref[...], axis=1, keepdims=True)
    deg = rcs - dcs + 1.0
    g = lamb * lax.rsqrt(deg)
    ids = i * x_ref.shape[0] + lax.broadcasted_iota(jnp.int32, (x_ref.shape[0], 1), 0)
    valid = ids < N
    g = jnp.where(valid, g, 0.0)
    u_ref[...] = g * s
    g_ref[...] = g
    sw_ref[...] = jnp.where(valid, 1.0 - dcs, 0.0)


def _dense(x_pad, w, bias, rc_t, dc_t):
    bs = 512
    grid = (N_PAD // bs,)
    return pl.pallas_call(
        _dense_body,
        grid=grid,
        in_specs=[
            pl.BlockSpec((bs, D), lambda i: (i, 0)),
            pl.BlockSpec((D, D), lambda i: (0, 0)),
            pl.BlockSpec((1, D), lambda i: (0, 0)),
            pl.BlockSpec((bs, NW), lambda i: (i, 0)),
            pl.BlockSpec((bs, NW), lambda i: (i, 0)),
        ],
        out_specs=(
            pl.BlockSpec((bs, D), lambda i: (i, 0)),
            pl.BlockSpec((bs, 1), lambda i: (i, 0)),
            pl.BlockSpec((bs, 1), lambda i: (i, 0)),
        ),
        out_shape=(
            jax.ShapeDtypeStruct((N_PAD, D), jnp.float32),
            jax.ShapeDtypeStruct((N_PAD, 1), jnp.float32),
            jax.ShapeDtypeStruct((N_PAD, 1), jnp.float32),
        ),
    )(x_pad, w, bias, rc_t, dc_t)


def _agg_body(u_hbm, g_hbm, row_hbm, col_hbm, acc_out, gsum_out,
              gtab, gsum, cidx0, ridx0, cidx1, ridx1, rows0, rows1,
              zbuf, acc, gsem0, gsem1, ssem):
    cid = lax.axis_index("c")
    sid = lax.axis_index("s")
    wid = sid * NC + cid

    pltpu.sync_copy(g_hbm, gtab)
    zeros = jnp.zeros((L,), jnp.float32)

    def zero_body(i, _):
        gsum[pl.ds(i * L, L)] = zeros
        return _

    lax.fori_loop(0, N_PAD // L, zero_body, 0)

    for r in range(16):
        for j in range(D // L):
            zbuf[r, pl.ds(j * L, L)] = zeros

    def zero_acc(i, _):
        pltpu.sync_copy(zbuf, acc.at[pl.ds(sid * ROWS_PT + i * 16, 16)])
        return _

    lax.fori_loop(0, ROWS_PT // 16, zero_acc, 0)
    plsc.subcore_barrier()

    bufs = ((cidx0, ridx0, rows0, gsem0), (cidx1, ridx1, rows1, gsem1))

    def load_idx(k, b):
        cb, rb, _, _ = bufs[b]
        base = sid * EPT1 + k * CHUNK
        pltpu.sync_copy(col_hbm.at[pl.ds(base, CHUNK)], cb)
        pltpu.sync_copy(row_hbm.at[pl.ds(base, CHUNK)], rb)

    def start_gather(b):
        cb, _, rw, sem = bufs[b]
        pltpu.async_copy(u_hbm.at[cb], rw, sem)

    def wait_gather(b):
        cb, _, rw, sem = bufs[b]
        pltpu.make_async_copy(u_hbm.at[cb], rw, sem).wait()

    def drain(b):
        cb, rb, rw, _ = bufs[b]
        desc = pltpu.async_copy(rw, acc.at[rb], ssem, add=True)

        def inner(j, _):
            cc = cb[pl.ds(j * L, L)]
            rr = rb[pl.ds(j * L, L)]
            gv = plsc.load_gather(gtab, [cc])
            plsc.addupdate_scatter(gsum, [rr], gv)
            return _

        lax.fori_loop(0, CHUNK // L, inner, 0)
        desc.wait()

    @pl.when(cid == 1)
    def _main():
        load_idx(0, 0)
        start_gather(0)

        def pair_body(i, _):
            k0 = 2 * i

            @pl.when(k0 + 1 < NCH1)
            def _1():
                load_idx(k0 + 1, 1)
                start_gather(1)

            wait_gather(0)
            drain(0)

            @pl.when(k0 + 2 < NCH1)
            def _2():
                load_idx(k0 + 2, 0)
                start_gather(0)

            @pl.when(k0 + 1 < NCH1)
            def _3():
                wait_gather(1)
                drain(1)

            return _

        lax.fori_loop(0, (NCH1 + 1) // 2, pair_body, 0)

    pltpu.sync_copy(gsum, gsum_out.at[wid])
    plsc.subcore_barrier()
    sl = pl.ds(sid * ROWS_PT, ROWS_PT)
    pltpu.sync_copy(acc.at[sl], acc_out.at[cid, sl])


def _agg(u, g_flat, row_p, col_p):
    return pl.kernel(
        _agg_body,
        out_type=(
            jax.ShapeDtypeStruct((NC, N_PAD, D), jnp.float32),
            jax.ShapeDtypeStruct((NW, N_PAD), jnp.float32),
        ),
        mesh=_mesh(),
        compiler_params=pltpu.CompilerParams(needs_layout_passes=False),
        scratch_types=[
            pltpu.VMEM((N_PAD,), jnp.float32),
            pltpu.VMEM((N_PAD,), jnp.float32),
            pltpu.VMEM((CHUNK,), jnp.int32),
            pltpu.VMEM((CHUNK,), jnp.int32),
            pltpu.VMEM((CHUNK,), jnp.int32),
            pltpu.VMEM((CHUNK,), jnp.int32),
            pltpu.VMEM((CHUNK, D), jnp.float32),
            pltpu.VMEM((CHUNK, D), jnp.float32),
            pltpu.VMEM((16, D), jnp.float32),
            pltpu.VMEM_SHARED((N_PAD, D), jnp.float32),
            pltpu.SemaphoreType.DMA,
            pltpu.SemaphoreType.DMA,
            pltpu.SemaphoreType.DMA,
        ],
    )(u, g_flat, row_p, col_p)


def _fin_body(p0_ref, p1_ref, gs_ref, u_ref, g_ref, sw_ref, o_ref):
    sw = sw_ref[...]
    num = p0_ref[...] + p1_ref[...] + sw * u_ref[...]
    den = jnp.sum(gs_ref[...], axis=1, keepdims=True) + sw * g_ref[...]
    sv = num / den
    ss = jnp.sum(sv * sv, axis=1, keepdims=True)
    out = sv / (1.0 + jnp.sqrt(jnp.maximum(1.0 - ss, MIN_NORM)))
    o_ref[...] = jnp.where(out > 0, out, 0.01 * out)


def _finish(p0, p1, gs_t, u, g, sw):
    bs = 400
    grid = (N // bs,)
    return pl.pallas_call(
        _fin_body,
        grid=grid,
        in_specs=[
            pl.BlockSpec((bs, D), lambda i: (i, 0)),
            pl.BlockSpec((bs, D), lambda i: (i, 0)),
            pl.BlockSpec((bs, NW), lambda i: (i, 0)),
            pl.BlockSpec((bs, D), lambda i: (i, 0)),
            pl.BlockSpec((bs, 1), lambda i: (i, 0)),
            pl.BlockSpec((bs, 1), lambda i: (i, 0)),
        ],
        out_specs=pl.BlockSpec((bs, D), lambda i: (i, 0)),
        out_shape=jax.ShapeDtypeStruct((N, D), jnp.float32),
    )(p0, p1, gs_t, u, g, sw)


def kernel(x, edge_index, W, bias):
    row = edge_index[0].astype(jnp.int32)
    col = edge_index[1].astype(jnp.int32)
    pad = jnp.full((E_PAD - E,), N, jnp.int32)
    row_p = jnp.concatenate([row, pad])
    col_p = jnp.concatenate([col, pad])

    rc, dc = _hist(row_p, col_p)

    x_pad = jnp.pad(x, ((0, N_PAD - N), (0, 0)))
    u, g, sw = _dense(x_pad, W, bias, rc.T, dc.T)

    acc, gsum = _agg(u, g.reshape(N_PAD), row_p, col_p)

    return _finish(acc[0], acc[1], gsum.T, u, g, sw)


# speedup vs baseline: 1.6336x; 1.6336x over previous
import numpy as np

import jax
import jax.numpy as jnp
from jax import lax
from jax.experimental import pallas as pl
from jax.experimental.pallas import tpu as pltpu
from jax.experimental.pallas import tpu_sc as plsc

N = 10000
D = 128
E = 320000
MIN_NORM = 1e-15
MAXNORM = 1.0 - 4e-3

NC = 2
NS = 16
NW = NC * NS
L = 16

N_PAD = 10240
EPT = 10240
E_PAD = EPT * NW
CHUNK = 64
NCH = EPT // CHUNK
ROWS_PT = N_PAD // NS

_PERM = np.arange(D).reshape(D // 32, 2, 16).transpose(0, 2, 1).reshape(D)


def _mesh():
    return plsc.VectorSubcoreMesh(
        core_axis_name="c", subcore_axis_name="s", num_cores=NC, num_subcores=NS
    )


def _hist_body(row_hbm, col_hbm, rc_out, dc_out, rbuf, cbuf, rc_priv, dc_priv):
    cid = lax.axis_index("c")
    sid = lax.axis_index("s")
    wid = sid * NC + cid

    zeros = jnp.zeros((L,), jnp.float32)

    def zero_body(i, _):
        rc_priv[pl.ds(i * L, L)] = zeros
        dc_priv[pl.ds(i * L, L)] = zeros
        return _

    lax.fori_loop(0, N_PAD // L, zero_body, 0)

    base = wid * EPT
    pltpu.sync_copy(row_hbm.at[pl.ds(base, EPT)], rbuf)
    pltpu.sync_copy(col_hbm.at[pl.ds(base, EPT)], cbuf)

    ones = jnp.full((L,), 1.0, jnp.float32)

    def body(i, _):
        r = rbuf[pl.ds(i * L, L)]
        c = cbuf[pl.ds(i * L, L)]
        plsc.addupdate_scatter(rc_priv, [r], ones)
        plsc.addupdate_scatter(dc_priv, [r], ones, mask=r == c)
        return _

    lax.fori_loop(0, EPT // L, body, 0)

    pltpu.sync_copy(rc_priv, rc_out.at[wid])
    pltpu.sync_copy(dc_priv, dc_out.at[wid])


def _hist(row_p, col_p):
    return pl.kernel(
        _hist_body,
        out_type=(
            jax.ShapeDtypeStruct((NW, N_PAD), jnp.float32),
            jax.ShapeDtypeStruct((NW, N_PAD), jnp.float32),
        ),
        mesh=_mesh(),
        compiler_params=pltpu.CompilerParams(needs_layout_passes=False),
        scratch_types=[
            pltpu.VMEM((EPT,), jnp.int32),
            pltpu.VMEM((EPT,), jnp.int32),
            pltpu.VMEM((N_PAD,), jnp.float32),
            pltpu.VMEM((N_PAD,), jnp.float32),
        ],
    )(row_p, col_p)


def _proj(v):
    n = jnp.maximum(jnp.sqrt(jnp.sum(v * v, axis=1, keepdims=True)), MIN_NORM)
    return jnp.where(n > MAXNORM, v * (MAXNORM / n), v)


def _dense_body(x_ref, w_ref, b_ref, rc_ref, dc_ref, u_ref, g_ref, sw_ref):
    i = pl.program_id(0)
    xb = x_ref[...]
    w = w_ref[...]
    h = lax.dot_general(xb, w, (((1,), (1,)), ((), ())),
                        preferred_element_type=jnp.float32)
    h = _proj(h)
    bias = b_ref[...]
    bn = jnp.maximum(jnp.sqrt(jnp.sum(bias * bias, axis=1, keepdims=True)),
                     MIN_NORM)
    b = jnp.tanh(bn) * bias / bn
    b = _proj(b)
    x2 = jnp.sum(h * h, axis=1, keepdims=True)
    y2 = jnp.sum(b * b, axis=1, keepdims=True)
    xy = jnp.sum(h * b, axis=1, keepdims=True)
    numer = (1.0 + 2.0 * xy + y2) * h + (1.0 - x2) * b
    denom = 1.0 + 2.0 * xy + x2 * y2
    h = numer / jnp.maximum(denom, MIN_NORM)
    h = _proj(h)
    s = 2.0 * h / (1.0 + jnp.sum(h * h, axis=1, keepdims=True))
    lamb = lax.rsqrt(jnp.maximum(1.0 - jnp.sum(s * s, axis=1, keepdims=True),
                                 MIN_NORM))
    rcs = jnp.sum(rc_ref[...], axis=1, keepdims=True)
    dcs = jnp.sum(dc_ref[...], axis=1, keepdims=True)
    deg = rcs - dcs + 1.0
    g = lamb * lax.rsqrt(deg)
    ids = i * x_ref.shape[0] + lax.broadcasted_iota(jnp.int32, (x_ref.shape[0], 1), 0)
    valid = ids < N
    g = jnp.where(valid, g, 0.0)
    u_ref[...] = g * s
    g_ref[...] = g
    sw_ref[...] = jnp.where(valid, 1.0 - dcs, 0.0)


def _dense(x_pad, w, bias, rc_t, dc_t):
    bs = 512
    grid = (N_PAD // bs,)
    return pl.pallas_call(
        _dense_body,
        grid=grid,
        in_specs=[
            pl.BlockSpec((bs, D), lambda i: (i, 0)),
            pl.BlockSpec((D, D), lambda i: (0, 0)),
            pl.BlockSpec((1, D), lambda i: (0, 0)),
            pl.BlockSpec((bs, NW), lambda i: (i, 0)),
            pl.BlockSpec((bs, NW), lambda i: (i, 0)),
        ],
        out_specs=(
            pl.BlockSpec((bs, D), lambda i: (i, 0)),
            pl.BlockSpec((bs, 1), lambda i: (i, 0)),
            pl.BlockSpec((bs, 1), lambda i: (i, 0)),
        ),
        out_shape=(
            jax.ShapeDtypeStruct((N_PAD, D), jnp.float32),
            jax.ShapeDtypeStruct((N_PAD, 1), jnp.float32),
            jax.ShapeDtypeStruct((N_PAD, 1), jnp.float32),
        ),
    )(x_pad, w, bias, rc_t, dc_t)


def _agg_body(u16_hbm, g_hbm, row_hbm, col_hbm, acc_out, gsum_out,
              gtab, gsum, cidx0, ridx0, cidx1, ridx1,
              rows16_0, rows16_1, rows32_0, rows32_1,
              zbuf, acc, gsem0, gsem1, ssem0, ssem1):
    cid = lax.axis_index("c")
    sid = lax.axis_index("s")
    wid = sid * NC + cid

    pltpu.sync_copy(g_hbm, gtab)
    zeros = jnp.zeros((L,), jnp.float32)

    def zero_body(i, _):
        gsum[pl.ds(i * L, L)] = zeros
        return _

    lax.fori_loop(0, N_PAD // L, zero_body, 0)

    for r in range(16):
        for j in range(D // L):
            zbuf[r, pl.ds(j * L, L)] = zeros

    def zero_acc(i, _):
        pltpu.sync_copy(zbuf, acc.at[pl.ds(sid * ROWS_PT + i * 16, 16)])
        return _

    lax.fori_loop(0, ROWS_PT // 16, zero_acc, 0)
    plsc.subcore_barrier()

    bufs = (
        (cidx0, ridx0, rows16_0, rows32_0, gsem0, ssem0),
        (cidx1, ridx1, rows16_1, rows32_1, gsem1, ssem1),
    )

    def load_idx(k, b):
        cb, rb, _, _, _, _ = bufs[b]
        base = wid * EPT + k * CHUNK
        pltpu.sync_copy(col_hbm.at[pl.ds(base, CHUNK)], cb)
        pltpu.sync_copy(row_hbm.at[pl.ds(base, CHUNK)], rb)

    def start_gather(b):
        cb, _, r16, _, sem, _ = bufs[b]
        pltpu.async_copy(u16_hbm.at[cb], r16, sem)

    def wait_gather(b):
        cb, _, r16, _, sem, _ = bufs[b]
        pltpu.make_async_copy(u16_hbm.at[cb], r16, sem).wait()

    def drain(b):
        cb, rb, r16, r32, _, sem = bufs[b]

        def conv(j, _):
            for k in range(D // 32):
                x = r16[j, pl.ds(32 * k, 32)]
                lo, hi = plsc.unpack(x, format=plsc.PackFormat.INTERLEAVED)
                r32[j, pl.ds(32 * k, L)] = lo
                r32[j, pl.ds(32 * k + L, L)] = hi
            return _

        lax.fori_loop(0, CHUNK, conv, 0)
        desc = pltpu.async_copy(r32, acc.at[rb], sem, add=True)

        def inner(j, _):
            cc = cb[pl.ds(j * L, L)]
            rr = rb[pl.ds(j * L, L)]
            gv = plsc.load_gather(gtab, [cc])
            plsc.addupdate_scatter(gsum, [rr], gv)
            return _

        lax.fori_loop(0, CHUNK // L, inner, 0)
        desc.wait()

    load_idx(0, 0)
    start_gather(0)

    def pair_body(i, _):
        k0 = 2 * i

        @pl.when(k0 + 1 < NCH)
        def _1():
            load_idx(k0 + 1, 1)
            start_gather(1)

        wait_gather(0)
        drain(0)

        @pl.when(k0 + 2 < NCH)
        def _2():
            load_idx(k0 + 2, 0)
            start_gather(0)

        @pl.when(k0 + 1 < NCH)
        def _3():
            wait_gather(1)
            drain(1)

        return _

    lax.fori_loop(0, (NCH + 1) // 2, pair_body, 0)

    pltpu.sync_copy(gsum, gsum_out.at[wid])
    plsc.subcore_barrier()
    sl = pl.ds(sid * ROWS_PT, ROWS_PT)
    pltpu.sync_copy(acc.at[sl], acc_out.at[cid, sl])


def _agg(u16, g_flat, row_p, col_p):
    return pl.kernel(
        _agg_body,
        out_type=(
            jax.ShapeDtypeStruct((NC, N_PAD, D), jnp.float32),
            jax.ShapeDtypeStruct((NW, N_PAD), jnp.float32),
        ),
        mesh=_mesh(),
        compiler_params=pltpu.CompilerParams(needs_layout_passes=False,
                                             use_tc_tiling_on_sc=False),
        scratch_types=[
            pltpu.VMEM((N_PAD,), jnp.float32),
            pltpu.VMEM((N_PAD,), jnp.float32),
            pltpu.VMEM((CHUNK,), jnp.int32),
            pltpu.VMEM((CHUNK,), jnp.int32),
            pltpu.VMEM((CHUNK,), jnp.int32),
            pltpu.VMEM((CHUNK,), jnp.int32),
            pltpu.VMEM((CHUNK, D), jnp.bfloat16),
            pltpu.VMEM((CHUNK, D), jnp.bfloat16),
            pltpu.VMEM((CHUNK, D), jnp.float32),
            pltpu.VMEM((CHUNK, D), jnp.float32),
            pltpu.VMEM((16, D), jnp.float32),
            pltpu.VMEM_SHARED((N_PAD, D), jnp.float32),
            pltpu.SemaphoreType.DMA,
            pltpu.SemaphoreType.DMA,
            pltpu.SemaphoreType.DMA,
            pltpu.SemaphoreType.DMA,
        ],
    )(u16, g_flat, row_p, col_p)


def _fin_body(p0_ref, p1_ref, gs_ref, u_ref, g_ref, sw_ref, o_ref):
    sw = sw_ref[...]
    num = p0_ref[...] + p1_ref[...] + sw * u_ref[...]
    den = jnp.sum(gs_ref[...], axis=1, keepdims=True) + sw * g_ref[...]
    sv = num / den
    ss = jnp.sum(sv * sv, axis=1, keepdims=True)
    out = sv / (1.0 + jnp.sqrt(jnp.maximum(1.0 - ss, MIN_NORM)))
    o_ref[...] = jnp.where(out > 0, out, 0.01 * out)


def _finish(p0, p1, gs_t, u, g, sw):
    bs = 400
    grid = (N // bs,)
    return pl.pallas_call(
        _fin_body,
        grid=grid,
        in_specs=[
            pl.BlockSpec((bs, D), lambda i: (i, 0)),
            pl.BlockSpec((bs, D), lambda i: (i, 0)),
            pl.BlockSpec((bs, NW), lambda i: (i, 0)),
            pl.BlockSpec((bs, D), lambda i: (i, 0)),
            pl.BlockSpec((bs, 1), lambda i: (i, 0)),
            pl.BlockSpec((bs, 1), lambda i: (i, 0)),
        ],
        out_specs=pl.BlockSpec((bs, D), lambda i: (i, 0)),
        out_shape=jax.ShapeDtypeStruct((N, D), jnp.float32),
    )(p0, p1, gs_t, u, g, sw)


def kernel(x, edge_index, W, bias):
    row = edge_index[0].astype(jnp.int32)
    col = edge_index[1].astype(jnp.int32)
    pad = jnp.full((E_PAD - E,), N, jnp.int32)
    row_p = jnp.concatenate([row, pad])
    col_p = jnp.concatenate([col, pad])

    rc, dc = _hist(row_p, col_p)

    x_pad = jnp.pad(x, ((0, N_PAD - N), (0, 0)))
    u, g, sw = _dense(x_pad, W, bias, rc.T, dc.T)

    u16 = u[:, _PERM].astype(jnp.bfloat16)
    acc, gsum = _agg(u16, g.reshape(N_PAD), row_p, col_p)

    return _finish(acc[0], acc[1], gsum.T, u, g, sw)


# speedup vs baseline: 1.7277x; 1.0576x over previous
import numpy as np

import jax
import jax.numpy as jnp
from jax import lax
from jax.experimental import pallas as pl
from jax.experimental.pallas import tpu as pltpu
from jax.experimental.pallas import tpu_sc as plsc

N = 10000
D = 128
E = 320000
MIN_NORM = 1e-15
MAXNORM = 1.0 - 4e-3

NC = 2
NS = 16
NW = NC * NS
L = 16

N_PAD = 10240
EPT = 10240
E_PAD = EPT * NW
CHUNK = 64
NCH = EPT // CHUNK
ROWS_PT = N_PAD // NS

_PERM = np.arange(D).reshape(D // 32, 2, 16).transpose(0, 2, 1).reshape(D)


def _mesh():
    return plsc.VectorSubcoreMesh(
        core_axis_name="c", subcore_axis_name="s", num_cores=NC, num_subcores=NS
    )


def _hist_body(row_hbm, col_hbm, rc_out, dc_out, rbuf, cbuf, rc_priv, dc_priv):
    cid = lax.axis_index("c")
    sid = lax.axis_index("s")
    wid = sid * NC + cid

    zeros = jnp.zeros((L,), jnp.float32)

    def zero_body(i, _):
        rc_priv[pl.ds(i * L, L)] = zeros
        dc_priv[pl.ds(i * L, L)] = zeros
        return _

    lax.fori_loop(0, N_PAD // L, zero_body, 0)

    base = wid * EPT
    pltpu.sync_copy(row_hbm.at[pl.ds(base, EPT)], rbuf)
    pltpu.sync_copy(col_hbm.at[pl.ds(base, EPT)], cbuf)

    ones = jnp.full((L,), 1.0, jnp.float32)

    def body(i, _):
        r = rbuf[pl.ds(i * L, L)]
        c = cbuf[pl.ds(i * L, L)]
        plsc.addupdate_scatter(rc_priv, [r], ones)
        plsc.addupdate_scatter(dc_priv, [r], ones, mask=r == c)
        return _

    lax.fori_loop(0, EPT // L, body, 0)

    pltpu.sync_copy(rc_priv, rc_out.at[wid])
    pltpu.sync_copy(dc_priv, dc_out.at[wid])


def _hist(row_p, col_p):
    return pl.kernel(
        _hist_body,
        out_type=(
            jax.ShapeDtypeStruct((NW, N_PAD), jnp.float32),
            jax.ShapeDtypeStruct((NW, N_PAD), jnp.float32),
        ),
        mesh=_mesh(),
        compiler_params=pltpu.CompilerParams(needs_layout_passes=False),
        scratch_types=[
            pltpu.VMEM((EPT,), jnp.int32),
            pltpu.VMEM((EPT,), jnp.int32),
            pltpu.VMEM((N_PAD,), jnp.float32),
            pltpu.VMEM((N_PAD,), jnp.float32),
        ],
    )(row_p, col_p)


def _proj(v):
    n = jnp.maximum(jnp.sqrt(jnp.sum(v * v, axis=1, keepdims=True)), MIN_NORM)
    return jnp.where(n > MAXNORM, v * (MAXNORM / n), v)


def _dense_body(x_ref, w_ref, b_ref, rc_ref, dc_ref, u_ref, g_ref, sw_ref):
    i = pl.program_id(0)
    xb = x_ref[...]
    w = w_ref[...]
    h = lax.dot_general(xb, w, (((1,), (1,)), ((), ())),
                        preferred_element_type=jnp.float32)
    h = _proj(h)
    bias = b_ref[...]
    bn = jnp.maximum(jnp.sqrt(jnp.sum(bias * bias, axis=1, keepdims=True)),
                     MIN_NORM)
    b = jnp.tanh(bn) * bias / bn
    b = _proj(b)
    x2 = jnp.sum(h * h, axis=1, keepdims=True)
    y2 = jnp.sum(b * b, axis=1, keepdims=True)
    xy = jnp.sum(h * b, axis=1, keepdims=True)
    numer = (1.0 + 2.0 * xy + y2) * h + (1.0 - x2) * b
    denom = 1.0 + 2.0 * xy + x2 * y2
    h = numer / jnp.maximum(denom, MIN_NORM)
    h = _proj(h)
    s = 2.0 * h / (1.0 + jnp.sum(h * h, axis=1, keepdims=True))
    lamb = lax.rsqrt(jnp.maximum(1.0 - jnp.sum(s * s, axis=1, keepdims=True),
                                 MIN_NORM))
    rcs = jnp.sum(rc_ref[...], axis=1, keepdims=True)
    dcs = jnp.sum(dc_ref[...], axis=1, keepdims=True)
    deg = rcs - dcs + 1.0
    g = lamb * lax.rsqrt(deg)
    ids = i * x_ref.shape[0] + lax.broadcasted_iota(jnp.int32, (x_ref.shape[0], 1), 0)
    valid = ids < N
    g = jnp.where(valid, g, 0.0)
    u_ref[...] = g * s
    g_ref[...] = g
    sw_ref[...] = jnp.where(valid, 1.0 - dcs, 0.0)


def _dense(x_pad, w, bias, rc_t, dc_t):
    bs = 512
    grid = (N_PAD // bs,)
    return pl.pallas_call(
        _dense_body,
        grid=grid,
        in_specs=[
            pl.BlockSpec((bs, D), lambda i: (i, 0)),
            pl.BlockSpec((D, D), lambda i: (0, 0)),
            pl.BlockSpec((1, D), lambda i: (0, 0)),
            pl.BlockSpec((bs, NW), lambda i: (i, 0)),
            pl.BlockSpec((bs, NW), lambda i: (i, 0)),
        ],
        out_specs=(
            pl.BlockSpec((bs, D), lambda i: (i, 0)),
            pl.BlockSpec((bs, 1), lambda i: (i, 0)),
            pl.BlockSpec((bs, 1), lambda i: (i, 0)),
        ),
        out_shape=(
            jax.ShapeDtypeStruct((N_PAD, D), jnp.float32),
            jax.ShapeDtypeStruct((N_PAD, 1), jnp.float32),
            jax.ShapeDtypeStruct((N_PAD, 1), jnp.float32),
        ),
    )(x_pad, w, bias, rc_t, dc_t)


def _agg_body(u16_hbm, g_hbm, row_hbm, col_hbm, acc_out, gsum_out,
              gtab, gsum, cidx0, ridx0, cidx1, ridx1,
              rows16_0, rows16_1, rows32_0, rows32_1,
              zbuf, acc, gsem0, gsem1, ssem0, ssem1):
    cid = lax.axis_index("c")
    sid = lax.axis_index("s")
    wid = sid * NC + cid

    pltpu.sync_copy(g_hbm, gtab)
    zeros = jnp.zeros((L,), jnp.float32)

    def zero_body(i, _):
        gsum[pl.ds(i * L, L)] = zeros
        return _

    lax.fori_loop(0, N_PAD // L, zero_body, 0)

    for r in range(16):
        for j in range(D // L):
            zbuf[r, pl.ds(j * L, L)] = zeros

    def zero_acc(i, _):
        pltpu.sync_copy(zbuf, acc.at[pl.ds(sid * ROWS_PT + i * 16, 16)])
        return _

    lax.fori_loop(0, ROWS_PT // 16, zero_acc, 0)
    plsc.subcore_barrier()

    bufs = (
        (cidx0, ridx0, rows16_0, rows32_0, gsem0, ssem0),
        (cidx1, ridx1, rows16_1, rows32_1, gsem1, ssem1),
    )

    def load_idx(k, b):
        cb, rb, _, _, _, _ = bufs[b]
        base = wid * EPT + k * CHUNK
        pltpu.sync_copy(col_hbm.at[pl.ds(base, CHUNK)], cb)
        pltpu.sync_copy(row_hbm.at[pl.ds(base // L, CHUNK // L)], rb)

    def start_gather(b):
        cb, _, r16, _, sem, _ = bufs[b]
        pltpu.async_copy(u16_hbm.at[cb], r16, sem)

    def wait_gather(b):
        cb, _, r16, _, sem, _ = bufs[b]
        pltpu.make_async_copy(u16_hbm.at[cb], r16, sem).wait()

    def drain(b):
        cb, rb, r16, r32, _, sem = bufs[b]
        descs = []
        for q in range(CHUNK // L):
            def conv(j, _, q=q):
                for k in range(D // 32):
                    x = r16[q * L + j, pl.ds(32 * k, 32)]
                    lo, hi = plsc.unpack(x, format=plsc.PackFormat.INTERLEAVED)
                    r32[q * L + j, pl.ds(32 * k, L)] = lo
                    r32[q * L + j, pl.ds(32 * k + L, L)] = hi
                return _

            lax.fori_loop(0, L, conv, 0)
            descs.append(pltpu.async_copy(
                r32.at[pl.ds(q * L, L)], acc.at[rb.at[q]], sem, add=True))

        def inner(j, _):
            cc = cb[pl.ds(j * L, L)]
            rr = rb[j, :]
            gv = plsc.load_gather(gtab, [cc])
            plsc.addupdate_scatter(gsum, [rr], gv)
            return _

        lax.fori_loop(0, CHUNK // L, inner, 0)
        for d in descs:
            d.wait()

    load_idx(0, 0)
    start_gather(0)

    def pair_body(i, _):
        k0 = 2 * i

        @pl.when(k0 + 1 < NCH)
        def _1():
            load_idx(k0 + 1, 1)
            start_gather(1)

        wait_gather(0)
        drain(0)

        @pl.when(k0 + 2 < NCH)
        def _2():
            load_idx(k0 + 2, 0)
            start_gather(0)

        @pl.when(k0 + 1 < NCH)
        def _3():
            wait_gather(1)
            drain(1)

        return _

    lax.fori_loop(0, (NCH + 1) // 2, pair_body, 0)

    pltpu.sync_copy(gsum, gsum_out.at[wid])
    plsc.subcore_barrier()
    sl = pl.ds(sid * ROWS_PT, ROWS_PT)
    pltpu.sync_copy(acc.at[sl], acc_out.at[cid, sl])


def _agg(u16, g_flat, row_p, col_p):
    return pl.kernel(
        _agg_body,
        out_type=(
            jax.ShapeDtypeStruct((NC, N_PAD, D), jnp.float32),
            jax.ShapeDtypeStruct((NW, N_PAD), jnp.float32),
        ),
        mesh=_mesh(),
        compiler_params=pltpu.CompilerParams(needs_layout_passes=False,
                                             use_tc_tiling_on_sc=False),
        scratch_types=[
            pltpu.VMEM((N_PAD,), jnp.float32),
            pltpu.VMEM((N_PAD,), jnp.float32),
            pltpu.VMEM((CHUNK,), jnp.int32),
            pltpu.VMEM((CHUNK // L, L), jnp.int32),
            pltpu.VMEM((CHUNK,), jnp.int32),
            pltpu.VMEM((CHUNK // L, L), jnp.int32),
            pltpu.VMEM((CHUNK, D), jnp.bfloat16),
            pltpu.VMEM((CHUNK, D), jnp.bfloat16),
            pltpu.VMEM((CHUNK, D), jnp.float32),
            pltpu.VMEM((CHUNK, D), jnp.float32),
            pltpu.VMEM((16, D), jnp.float32),
            pltpu.VMEM_SHARED((N_PAD, D), jnp.float32),
            pltpu.SemaphoreType.DMA,
            pltpu.SemaphoreType.DMA,
            pltpu.SemaphoreType.DMA,
            pltpu.SemaphoreType.DMA,
        ],
    )(u16, g_flat, row_p.reshape(E_PAD // L, L), col_p)


def _fin_body(p0_ref, p1_ref, gs_ref, u_ref, g_ref, sw_ref, o_ref):
    sw = sw_ref[...]
    num = p0_ref[...] + p1_ref[...] + sw * u_ref[...]
    den = jnp.sum(gs_ref[...], axis=1, keepdims=True) + sw * g_ref[...]
    sv = num / den
    ss = jnp.sum(sv * sv, axis=1, keepdims=True)
    out = sv / (1.0 + jnp.sqrt(jnp.maximum(1.0 - ss, MIN_NORM)))
    o_ref[...] = jnp.where(out > 0, out, 0.01 * out)


def _finish(p0, p1, gs_t, u, g, sw):
    bs = 400
    grid = (N // bs,)
    return pl.pallas_call(
        _fin_body,
        grid=grid,
        in_specs=[
            pl.BlockSpec((bs, D), lambda i: (i, 0)),
            pl.BlockSpec((bs, D), lambda i: (i, 0)),
            pl.BlockSpec((bs, NW), lambda i: (i, 0)),
            pl.BlockSpec((bs, D), lambda i: (i, 0)),
            pl.BlockSpec((bs, 1), lambda i: (i, 0)),
            pl.BlockSpec((bs, 1), lambda i: (i, 0)),
        ],
        out_specs=pl.BlockSpec((bs, D), lambda i: (i, 0)),
        out_shape=jax.ShapeDtypeStruct((N, D), jnp.float32),
    )(p0, p1, gs_t, u, g, sw)


def kernel(x, edge_index, W, bias):
    row = edge_index[0].astype(jnp.int32)
    col = edge_index[1].astype(jnp.int32)
    pad = jnp.full((E_PAD - E,), N, jnp.int32)
    row_p = jnp.concatenate([row, pad])
    col_p = jnp.concatenate([col, pad])

    rc, dc = _hist(row_p, col_p)

    x_pad = jnp.pad(x, ((0, N_PAD - N), (0, 0)))
    u, g, sw = _dense(x_pad, W, bias, rc.T, dc.T)

    u16 = u[:, _PERM].astype(jnp.bfloat16)
    acc, gsum = _agg(u16, g.reshape(N_PAD), row_p, col_p)

    return _finish(acc[0], acc[1], gsum.T, u, g, sw)
